# baseline (speedup 1.0000x reference)
import functools

import jax
import jax.numpy as jnp
import numpy as np
from jax import lax
from jax.experimental import pallas as pl
from jax.experimental.pallas import tpu as pltpu
from jax.experimental.pallas import tpu_sc as plsc

_NC = 2
_NS = 16


def _pack_bf16_pairs(val):
    half = val.shape[1] // 2
    au = pltpu.bitcast(val[:, :half].astype(jnp.bfloat16), jnp.uint16)
    bu = pltpu.bitcast(val[:, half:].astype(jnp.bfloat16), jnp.uint16)
    return au.astype(jnp.int32) | jnp.left_shift(bu.astype(jnp.int32), 16)


def _edge_body(ea_ref, we_ref, be_ref, wm_ref, bm_ref, o_ref):
    e = jnp.maximum(
        jnp.dot(ea_ref[...], we_ref[...], preferred_element_type=jnp.float32)
        + be_ref[...],
        0.0,
    )
    o_ref[...] = _pack_bf16_pairs(
        jnp.dot(e, wm_ref[...], preferred_element_type=jnp.float32)
        + bm_ref[...]
    )


def _node_body(x_ref, wn_ref, bn_ref, wm0_ref, h_ref, hw_ref):
    h = jnp.maximum(
        jnp.dot(x_ref[...], wn_ref[...], preferred_element_type=jnp.float32)
        + bn_ref[...],
        0.0,
    )
    h_ref[...] = h
    hw_ref[...] = jnp.dot(h, wm0_ref[...], preferred_element_type=jnp.float32)


def _update_body(agg_ref, h_ref, wu_ref, bu_ref, wm_ref, ho_ref, hw_ref):
    agg = agg_ref[0] + agg_ref[1]
    upd = jnp.maximum(
        jnp.dot(agg, wu_ref[...], preferred_element_type=jnp.float32)
        + bu_ref[...],
        0.0,
    )
    hn = h_ref[...] + upd
    ho_ref[...] = hn
    hw_ref[...] = jnp.dot(hn, wm_ref[...], preferred_element_type=jnp.float32)


def _pool_body(h_ref, b_ref, sum_ref, cnt_ref, *, num_graphs):
    i = pl.program_id(0)
    blk = h_ref.shape[0]
    onehot = (
        b_ref[...]
        == lax.broadcasted_iota(jnp.int32, (blk, num_graphs), 1)
    ).astype(jnp.float32)
    psum = lax.dot_general(
        onehot, h_ref[...], (((0,), (0,)), ((), ())),
        preferred_element_type=jnp.float32,
    )
    pcnt = lax.dot_general(
        onehot, jnp.ones_like(h_ref[...]), (((0,), (0,)), ((), ())),
        preferred_element_type=jnp.float32,
    )

    @pl.when(i == 0)
    def _():
        sum_ref[...] = psum
        cnt_ref[...] = pcnt

    @pl.when(i > 0)
    def _():
        sum_ref[...] += psum
        cnt_ref[...] += pcnt


def _silu(v):
    return v * jax.nn.sigmoid(v)


def _graph_body(
    t_ref, sum_ref, cnt_ref,
    wt1s_ref, wt1c_ref, bt1_ref, wt2_ref, bt2_ref,
    wg1_ref, bg1_ref, wg2_ref, bg2_ref, tg_ref, *, tdim
):
    half = tdim // 2
    j = lax.broadcasted_iota(jnp.int32, (1, half), 1).astype(jnp.float32)
    freqs = jnp.exp(-np.log(10000.0) * j / half)
    args = t_ref[...].astype(jnp.float32) * freqs
    pre = (
        jnp.dot(jnp.sin(args), wt1s_ref[...], preferred_element_type=jnp.float32)
        + jnp.dot(jnp.cos(args), wt1c_ref[...], preferred_element_type=jnp.float32)
        + bt1_ref[...]
    )
    tf = (
        jnp.dot(_silu(pre), wt2_ref[...], preferred_element_type=jnp.float32)
        + bt2_ref[...]
    )
    gfeat = sum_ref[...] / jnp.maximum(cnt_ref[...], 1.0)
    pre2 = (
        jnp.dot(gfeat, wg1_ref[...], preferred_element_type=jnp.float32)
        + bg1_ref[...]
    )
    gf = (
        jnp.dot(_silu(pre2), wg2_ref[...], preferred_element_type=jnp.float32)
        + bg2_ref[...]
    )
    tg_ref[...] = tf + gf


def _final_body(
    h_ref, b_ref, tg_ref,
    wn1a_ref, wn1b_ref, bn1_ref, wn2_ref, bn2_ref, wn3_ref, bn3_ref,
    out_ref, *, num_graphs
):
    blk = h_ref.shape[0]
    onehot = (
        b_ref[...]
        == lax.broadcasted_iota(jnp.int32, (blk, num_graphs), 1)
    ).astype(jnp.float32)
    tfeat = jnp.dot(onehot, tg_ref[...], preferred_element_type=jnp.float32)
    u = _silu(
        jnp.dot(h_ref[...], wn1a_ref[...], preferred_element_type=jnp.float32)
        + jnp.dot(tfeat, wn1b_ref[...], preferred_element_type=jnp.float32)
        + bn1_ref[...]
    )
    u = _silu(
        jnp.dot(u, wn2_ref[...], preferred_element_type=jnp.float32)
        + bn2_ref[...]
    )
    out_ref[...] = (
        jnp.dot(u, wn3_ref[...], preferred_element_type=jnp.float32)
        + bn3_ref[...]
    )


@functools.cache
def _make_agg_kernel(n_nodes, n_edges, hid):
    nw = _NC * _NS
    epw = n_edges // nw
    chunk = 40
    niter = epw // chunk
    nvec = hid // 16
    rows_main = ((n_nodes // _NS) + 7) // 8 * 8
    rows_last = n_nodes - (_NS - 1) * rows_main
    assert rows_last > 0 and rows_last % 8 == 0

    mesh = plsc.VectorSubcoreMesh(core_axis_name="c", subcore_axis_name="s")

    nbuf = 4
    assert niter % nbuf == 2 and niter >= 2 * nbuf
    nquad = (niter - 2) // nbuf

    @functools.partial(
        pl.kernel,
        out_type=jax.ShapeDtypeStruct((_NC, n_nodes, hid), jnp.float32),
        mesh=mesh,
        scratch_types=(
            [pltpu.VMEM((chunk,), jnp.int32)] * nbuf
            + [pltpu.VMEM((chunk,), jnp.int32)] * nbuf
            + [pltpu.VMEM((chunk, hid // 2), jnp.int32)] * nbuf
            + [pltpu.VMEM((chunk, hid), jnp.float32)] * nbuf
            + [pltpu.VMEM_SHARED((n_nodes, hid), jnp.float32)]
            + [pltpu.SemaphoreType.DMA] * (4 * nbuf)
        ),
    )
    def agg_kernel(hw_hbm, ew_hbm, src_hbm, dst_hbm, out_hbm, *rest):
        srcs = rest[0:nbuf]
        dsts = rest[nbuf:2 * nbuf]
        ews = rest[2 * nbuf:3 * nbuf]
        gats = rest[3 * nbuf:4 * nbuf]
        acc_sh = rest[4 * nbuf]
        sps = rest[4 * nbuf + 1:5 * nbuf + 1]
        sds = rest[5 * nbuf + 1:6 * nbuf + 1]
        sgs = rest[6 * nbuf + 1:7 * nbuf + 1]
        sss = rest[7 * nbuf + 1:8 * nbuf + 1]
        cid = lax.axis_index("c")
        sid = lax.axis_index("s")
        wid = cid * _NS + sid
        ebase = wid * epw

        def coff(c):
            if isinstance(c, int) and c < niter:
                return ebase + c * chunk
            return ebase + jnp.minimum(c, niter - 1) * chunk

        def pre_issue(c, b):
            off = coff(c)
            pltpu.async_copy(src_hbm.at[pl.ds(off, chunk)], srcs[b], sps[b])
            pltpu.async_copy(ew_hbm.at[pl.ds(off, chunk)], ews[b], sps[b])

        def pre_wait(c, b):
            off = coff(c)
            pltpu.make_async_copy(src_hbm.at[pl.ds(off, chunk)], srcs[b], sps[b]).wait()
            pltpu.make_async_copy(ew_hbm.at[pl.ds(off, chunk)], ews[b], sps[b]).wait()

        def dst_issue(c, b):
            pltpu.async_copy(dst_hbm.at[pl.ds(coff(c), chunk)], dsts[b], sds[b])

        def dst_wait(c, b):
            pltpu.make_async_copy(
                dst_hbm.at[pl.ds(coff(c), chunk)], dsts[b], sds[b]).wait()

        def gather_issue(b):
            pltpu.async_copy(hw_hbm.at[srcs[b]], gats[b], sgs[b])

        def gather_wait(b):
            pltpu.make_async_copy(hw_hbm.at[srcs[b]], gats[b], sgs[b]).wait()

        def scatter_issue(b):
            pltpu.async_copy(gats[b], acc_sh.at[dsts[b]], sss[b], add=True)

        def scatter_wait(b):
            pltpu.make_async_copy(gats[b], acc_sh.at[dsts[b]], sss[b]).wait()

        def compute(b):
            g_ref, e_ref = gats[b], ews[b]

            himask = jnp.int32(-65536)

            def edge(i, _):
                for g in range(hid // 32):
                    we_ = e_ref[i, pl.ds(g * 16, 16)]
                    ae = lax.bitcast_convert_type(
                        jnp.left_shift(we_, 16), jnp.float32)
                    be = lax.bitcast_convert_type(we_ & himask, jnp.float32)
                    slo = pl.ds(g * 32, 16)
                    shi = pl.ds(g * 32 + 16, 16)
                    g_ref[i, slo] = jnp.maximum(g_ref[i, slo] + ae, 0.0)
                    g_ref[i, shi] = jnp.maximum(g_ref[i, shi] + be, 0.0)
                return 0
            lax.fori_loop(0, chunk, edge, 0, unroll=4)

        for b in range(nbuf):
            pre_issue(b, b)
        dst_issue(0, 0)
        dst_issue(1, 1)

        z0 = gats[0]

        def zrow(i, _):
            for j in range(nvec):
                z0[i, pl.ds(j * 16, 16)] = jnp.zeros((16,), jnp.float32)
            return 0
        lax.fori_loop(0, chunk, zrow, 0)
        base_r = sid * rows_main

        def span_copy(nrows, fn):
            full, rem = divmod(nrows, chunk)
            for k in range(full):
                fn(k * chunk, chunk)
            if rem:
                fn(full * chunk, rem)

        def zero_fn(r0, cnt):
            pltpu.sync_copy(z0.at[pl.ds(0, cnt)],
                            acc_sh.at[pl.ds(base_r + r0, cnt)])

        @pl.when(sid < _NS - 1)
        def _():
            span_copy(rows_main, zero_fn)

        @pl.when(sid == _NS - 1)
        def _():
            span_copy(rows_last, zero_fn)

        plsc.subcore_barrier()

        pre_wait(0, 0)
        pre_wait(1, 1)
        pre_wait(2, 2)
        gather_issue(0)
        gather_issue(1)

        def quad(i, _):
            cb = nbuf * i
            for k in range(nbuf):
                c = cb + k
                b = k
                b2 = (k + 2) % nbuf
                b3 = (k + 3) % nbuf
                gather_wait(b)
                compute(b)
                dst_wait(c, b)
                scatter_issue(b)
                pre_issue(c + 4, b)
                pre_wait(c + 3, b3)
                if k < 2:
                    @pl.when(i > 0)
                    def _():
                        scatter_wait(b2)
                else:
                    scatter_wait(b2)
                dst_issue(c + 2, b2)
                gather_issue(b2)
            return 0

        lax.fori_loop(0, nquad, quad, 0)

        gather_wait(0)
        compute(0)
        dst_wait(niter - 2, 0)
        scatter_issue(0)
        scatter_wait(2)

        gather_wait(1)
        compute(1)
        dst_wait(niter - 1, 1)
        scatter_issue(1)
        scatter_wait(3)

        pre_wait(niter + 1, 3)
        scatter_wait(0)
        scatter_wait(1)
        plsc.subcore_barrier()

        def out_fn(r0, cnt):
            pltpu.sync_copy(acc_sh.at[pl.ds(base_r + r0, cnt)],
                            out_hbm.at[cid, pl.ds(base_r + r0, cnt)])

        @pl.when(sid < _NS - 1)
        def _():
            span_copy(rows_main, out_fn)

        @pl.when(sid == _NS - 1)
        def _():
            span_copy(rows_last, out_fn)

    return agg_kernel


def kernel(x, edge_index, edge_attr, batch, t, params):
    n, atom = x.shape
    e_cnt, bond = edge_attr.shape
    hid = params["W_node"].shape[1]
    g_cnt = t.shape[0]
    tdim = params["W_t1"].shape[0]
    nlayers = params["W_msg"].shape[0]

    blk_e = 2000
    blk_n = 2000

    f32 = jnp.float32
    b_node = params["b_node"].reshape(1, hid)
    b_edge = params["b_edge"].reshape(1, hid)
    blocks = np.arange(hid).reshape(hid // 32, 2, 16)
    perm = np.concatenate([blocks[:, 0, :].ravel(), blocks[:, 1, :].ravel()])
    w_msg_p = params["W_msg"][:, :, perm]
    b_msg_p = params["b_msg"][:, perm]
    b_upd = params["b_upd"]
    src = edge_index[0]
    dst = edge_index[1]
    batch2 = batch.reshape(n, 1)
    t2 = t.reshape(g_cnt, 1)

    edge_grid = e_cnt // blk_e
    edge_call = pl.pallas_call(
        _edge_body,
        grid=(edge_grid,),
        in_specs=[
            pl.BlockSpec((blk_e, bond), lambda i: (i, 0)),
            pl.BlockSpec((bond, hid), lambda i: (0, 0)),
            pl.BlockSpec((1, hid), lambda i: (0, 0)),
            pl.BlockSpec((hid, hid), lambda i: (0, 0)),
            pl.BlockSpec((1, hid), lambda i: (0, 0)),
        ],
        out_specs=pl.BlockSpec((blk_e, hid // 2), lambda i: (i, 0)),
        out_shape=jax.ShapeDtypeStruct((e_cnt, hid // 2), jnp.int32),
    )

    def edge_terms(l):
        return edge_call(
            edge_attr, params["W_edge"], b_edge,
            w_msg_p[l], b_msg_p[l].reshape(1, hid),
        )

    ew_cur = edge_terms(0)

    node_grid = n // blk_n
    h, hw = pl.pallas_call(
        _node_body,
        grid=(node_grid,),
        in_specs=[
            pl.BlockSpec((blk_n, atom), lambda i: (i, 0)),
            pl.BlockSpec((atom, hid), lambda i: (0, 0)),
            pl.BlockSpec((1, hid), lambda i: (0, 0)),
            pl.BlockSpec((hid, hid), lambda i: (0, 0)),
        ],
        out_specs=[pl.BlockSpec((blk_n, hid), lambda i: (i, 0))] * 2,
        out_shape=[jax.ShapeDtypeStruct((n, hid), f32)] * 2,
    )(x, params["W_node"], b_node, params["W_msg"][0])

    agg_kernel = _make_agg_kernel(n, e_cnt, hid)
    update_call = pl.pallas_call(
        _update_body,
        grid=(node_grid,),
        in_specs=[
            pl.BlockSpec((_NC, blk_n, hid), lambda i: (0, i, 0)),
            pl.BlockSpec((blk_n, hid), lambda i: (i, 0)),
            pl.BlockSpec((hid, hid), lambda i: (0, 0)),
            pl.BlockSpec((1, hid), lambda i: (0, 0)),
            pl.BlockSpec((hid, hid), lambda i: (0, 0)),
        ],
        out_specs=[pl.BlockSpec((blk_n, hid), lambda i: (i, 0))] * 2,
        out_shape=[jax.ShapeDtypeStruct((n, hid), f32)] * 2,
    )

    for l in range(nlayers):
        agg2 = agg_kernel(hw, ew_cur, src, dst)
        if l + 1 < nlayers:
            ew_cur = edge_terms(l + 1)
        wm_next = params["W_msg"][l + 1] if l + 1 < nlayers else params["W_msg"][0]
        h, hw = update_call(
            agg2, h, params["W_upd"][l], b_upd[l].reshape(1, hid), wm_next
        )

    sums, cnts = pl.pallas_call(
        functools.partial(_pool_body, num_graphs=g_cnt),
        grid=(node_grid,),
        in_specs=[
            pl.BlockSpec((blk_n, hid), lambda i: (i, 0)),
            pl.BlockSpec((blk_n, 1), lambda i: (i, 0)),
        ],
        out_specs=[pl.BlockSpec((g_cnt, hid), lambda i: (0, 0))] * 2,
        out_shape=[jax.ShapeDtypeStruct((g_cnt, hid), f32)] * 2,
        compiler_params=pltpu.CompilerParams(
            dimension_semantics=("arbitrary",)
        ),
    )(h, batch2)

    half = tdim // 2
    tg = pl.pallas_call(
        functools.partial(_graph_body, tdim=tdim),
        in_specs=[pl.BlockSpec(a.shape, lambda: tuple([0] * a.ndim)) for a in (
            t2, sums, cnts,
            params["W_t1"][:half], params["W_t1"][half:],
            params["b_t1"].reshape(1, hid), params["W_t2"],
            params["b_t2"].reshape(1, hid),
            params["W_g1"], params["b_g1"].reshape(1, hid),
            params["W_g2"], params["b_g2"].reshape(1, hid),
        )],
        out_specs=pl.BlockSpec((g_cnt, hid), lambda: (0, 0)),
        out_shape=jax.ShapeDtypeStruct((g_cnt, hid), f32),
    )(
        t2, sums, cnts,
        params["W_t1"][:half], params["W_t1"][half:],
        params["b_t1"].reshape(1, hid), params["W_t2"],
        params["b_t2"].reshape(1, hid),
        params["W_g1"], params["b_g1"].reshape(1, hid),
        params["W_g2"], params["b_g2"].reshape(1, hid),
    )

    out = pl.pallas_call(
        functools.partial(_final_body, num_graphs=g_cnt),
        grid=(node_grid,),
        in_specs=[
            pl.BlockSpec((blk_n, hid), lambda i: (i, 0)),
            pl.BlockSpec((blk_n, 1), lambda i: (i, 0)),
            pl.BlockSpec((g_cnt, hid), lambda i: (0, 0)),
            pl.BlockSpec((hid, hid), lambda i: (0, 0)),
            pl.BlockSpec((hid, hid), lambda i: (0, 0)),
            pl.BlockSpec((1, hid), lambda i: (0, 0)),
            pl.BlockSpec((hid, hid), lambda i: (0, 0)),
            pl.BlockSpec((1, hid), lambda i: (0, 0)),
            pl.BlockSpec((hid, atom), lambda i: (0, 0)),
            pl.BlockSpec((1, atom), lambda i: (0, 0)),
        ],
        out_specs=pl.BlockSpec((blk_n, atom), lambda i: (i, 0)),
        out_shape=jax.ShapeDtypeStruct((n, atom), f32),
    )(
        h, batch2, tg,
        params["W_n1"][:hid], params["W_n1"][hid:],
        params["b_n1"].reshape(1, hid),
        params["W_n2"], params["b_n2"].reshape(1, hid),
        params["W_n3"], params["b_n3"].reshape(1, atom),
    )
    return out


# speedup vs baseline: 1.1836x; 1.1836x over previous
import functools

import jax
import jax.numpy as jnp
import numpy as np
from jax import lax
from jax.experimental import pallas as pl
from jax.experimental.pallas import tpu as pltpu
from jax.experimental.pallas import tpu_sc as plsc

_NC = 2
_NS = 16


def _pack_bf16_pairs(val):
    half = val.shape[1] // 2
    au = pltpu.bitcast(val[:, :half].astype(jnp.bfloat16), jnp.uint16)
    bu = pltpu.bitcast(val[:, half:].astype(jnp.bfloat16), jnp.uint16)
    return au.astype(jnp.int32) | jnp.left_shift(bu.astype(jnp.int32), 16)


def _edge_body(ea_ref, we_ref, be_ref, wm_ref, bm_ref, o0_ref, o1_ref, o2_ref):
    e = jnp.maximum(
        jnp.dot(ea_ref[...], we_ref[...], preferred_element_type=jnp.float32)
        + be_ref[...],
        0.0,
    )
    for l, o_ref in enumerate((o0_ref, o1_ref, o2_ref)):
        o_ref[...] = _pack_bf16_pairs(
            jnp.dot(e, wm_ref[l], preferred_element_type=jnp.float32)
            + bm_ref[l : l + 1]
        )


def _node_body(x_ref, wn_ref, bn_ref, wm0_ref, h_ref, hw_ref):
    h = jnp.maximum(
        jnp.dot(x_ref[...], wn_ref[...], preferred_element_type=jnp.float32)
        + bn_ref[...],
        0.0,
    )
    h_ref[...] = h
    hw_ref[...] = jnp.dot(h, wm0_ref[...], preferred_element_type=jnp.float32)


def _update_pool_body(agg_ref, h_ref, wu_ref, bu_ref, b_ref,
                      ho_ref, sum_ref, cnt_ref, *, num_graphs):
    i = pl.program_id(0)
    agg = agg_ref[0] + agg_ref[1]
    upd = jnp.maximum(
        jnp.dot(agg, wu_ref[...], preferred_element_type=jnp.float32)
        + bu_ref[...],
        0.0,
    )
    hn = h_ref[...] + upd
    ho_ref[...] = hn
    blk = h_ref.shape[0]
    onehot = (
        b_ref[...]
        == lax.broadcasted_iota(jnp.int32, (blk, num_graphs), 1)
    ).astype(jnp.float32)
    psum = lax.dot_general(
        onehot, hn, (((0,), (0,)), ((), ())),
        preferred_element_type=jnp.float32,
    )
    pcnt = lax.dot_general(
        onehot, jnp.ones_like(hn), (((0,), (0,)), ((), ())),
        preferred_element_type=jnp.float32,
    )

    @pl.when(i == 0)
    def _():
        sum_ref[...] = psum
        cnt_ref[...] = pcnt

    @pl.when(i > 0)
    def _():
        sum_ref[...] += psum
        cnt_ref[...] += pcnt


def _update_body(agg_ref, h_ref, wu_ref, bu_ref, wm_ref, ho_ref, hw_ref):
    agg = agg_ref[0] + agg_ref[1]
    upd = jnp.maximum(
        jnp.dot(agg, wu_ref[...], preferred_element_type=jnp.float32)
        + bu_ref[...],
        0.0,
    )
    hn = h_ref[...] + upd
    ho_ref[...] = hn
    hw_ref[...] = jnp.dot(hn, wm_ref[...], preferred_element_type=jnp.float32)


def _pool_body(h_ref, b_ref, sum_ref, cnt_ref, *, num_graphs):
    i = pl.program_id(0)
    blk = h_ref.shape[0]
    onehot = (
        b_ref[...]
        == lax.broadcasted_iota(jnp.int32, (blk, num_graphs), 1)
    ).astype(jnp.float32)
    psum = lax.dot_general(
        onehot, h_ref[...], (((0,), (0,)), ((), ())),
        preferred_element_type=jnp.float32,
    )
    pcnt = lax.dot_general(
        onehot, jnp.ones_like(h_ref[...]), (((0,), (0,)), ((), ())),
        preferred_element_type=jnp.float32,
    )

    @pl.when(i == 0)
    def _():
        sum_ref[...] = psum
        cnt_ref[...] = pcnt

    @pl.when(i > 0)
    def _():
        sum_ref[...] += psum
        cnt_ref[...] += pcnt


def _silu(v):
    return v * jax.nn.sigmoid(v)


def _graph_body(
    t_ref, sum_ref, cnt_ref,
    wt1s_ref, wt1c_ref, bt1_ref, wt2_ref, bt2_ref,
    wg1_ref, bg1_ref, wg2_ref, bg2_ref, tg_ref, *, tdim
):
    half = tdim // 2
    j = lax.broadcasted_iota(jnp.int32, (1, half), 1).astype(jnp.float32)
    freqs = jnp.exp(-np.log(10000.0) * j / half)
    args = t_ref[...].astype(jnp.float32) * freqs
    pre = (
        jnp.dot(jnp.sin(args), wt1s_ref[...], preferred_element_type=jnp.float32)
        + jnp.dot(jnp.cos(args), wt1c_ref[...], preferred_element_type=jnp.float32)
        + bt1_ref[...]
    )
    tf = (
        jnp.dot(_silu(pre), wt2_ref[...], preferred_element_type=jnp.float32)
        + bt2_ref[...]
    )
    gfeat = sum_ref[...] / jnp.maximum(cnt_ref[...], 1.0)
    pre2 = (
        jnp.dot(gfeat, wg1_ref[...], preferred_element_type=jnp.float32)
        + bg1_ref[...]
    )
    gf = (
        jnp.dot(_silu(pre2), wg2_ref[...], preferred_element_type=jnp.float32)
        + bg2_ref[...]
    )
    tg_ref[...] = tf + gf


def _final_body(
    h_ref, b_ref, tg_ref,
    wn1a_ref, wn1b_ref, bn1_ref, wn2_ref, bn2_ref, wn3_ref, bn3_ref,
    out_ref, *, num_graphs
):
    blk = h_ref.shape[0]
    onehot = (
        b_ref[...]
        == lax.broadcasted_iota(jnp.int32, (blk, num_graphs), 1)
    ).astype(jnp.float32)
    tfeat = jnp.dot(onehot, tg_ref[...], preferred_element_type=jnp.float32)
    u = _silu(
        jnp.dot(h_ref[...], wn1a_ref[...], preferred_element_type=jnp.float32)
        + jnp.dot(tfeat, wn1b_ref[...], preferred_element_type=jnp.float32)
        + bn1_ref[...]
    )
    u = _silu(
        jnp.dot(u, wn2_ref[...], preferred_element_type=jnp.float32)
        + bn2_ref[...]
    )
    out_ref[...] = (
        jnp.dot(u, wn3_ref[...], preferred_element_type=jnp.float32)
        + bn3_ref[...]
    )


@functools.cache
def _make_agg_kernel(n_nodes, n_edges, hid):
    nw = _NC * _NS
    epw = n_edges // nw
    chunk = 40
    niter = epw // chunk
    nvec = hid // 16
    rows_main = ((n_nodes // _NS) + 7) // 8 * 8
    rows_last = n_nodes - (_NS - 1) * rows_main
    assert rows_last > 0 and rows_last % 8 == 0

    mesh = plsc.VectorSubcoreMesh(core_axis_name="c", subcore_axis_name="s")

    nbuf = 4
    assert niter % nbuf == 2 and niter >= 2 * nbuf
    nquad = (niter - 2) // nbuf

    @functools.partial(
        pl.kernel,
        out_type=jax.ShapeDtypeStruct((_NC, n_nodes, hid), jnp.float32),
        mesh=mesh,
        scratch_types=(
            [pltpu.VMEM((chunk,), jnp.int32)] * nbuf
            + [pltpu.VMEM((chunk,), jnp.int32)] * nbuf
            + [pltpu.VMEM((chunk, hid // 2), jnp.int32)] * nbuf
            + [pltpu.VMEM((chunk, hid), jnp.float32)] * nbuf
            + [pltpu.VMEM_SHARED((n_nodes, hid), jnp.float32)]
            + [pltpu.SemaphoreType.DMA] * (4 * nbuf)
        ),
    )
    def agg_kernel(hw_hbm, ew_hbm, src_hbm, dst_hbm, out_hbm, *rest):
        srcs = rest[0:nbuf]
        dsts = rest[nbuf:2 * nbuf]
        ews = rest[2 * nbuf:3 * nbuf]
        gats = rest[3 * nbuf:4 * nbuf]
        acc_sh = rest[4 * nbuf]
        sps = rest[4 * nbuf + 1:5 * nbuf + 1]
        sds = rest[5 * nbuf + 1:6 * nbuf + 1]
        sgs = rest[6 * nbuf + 1:7 * nbuf + 1]
        sss = rest[7 * nbuf + 1:8 * nbuf + 1]
        cid = lax.axis_index("c")
        sid = lax.axis_index("s")
        wid = cid * _NS + sid
        ebase = wid * epw

        def coff(c):
            if isinstance(c, int) and c < niter:
                return ebase + c * chunk
            return ebase + jnp.minimum(c, niter - 1) * chunk

        def pre_issue(c, b):
            off = coff(c)
            pltpu.async_copy(src_hbm.at[pl.ds(off, chunk)], srcs[b], sps[b])
            pltpu.async_copy(ew_hbm.at[pl.ds(off, chunk)], ews[b], sps[b])

        def pre_wait(c, b):
            off = coff(c)
            pltpu.make_async_copy(src_hbm.at[pl.ds(off, chunk)], srcs[b], sps[b]).wait()
            pltpu.make_async_copy(ew_hbm.at[pl.ds(off, chunk)], ews[b], sps[b]).wait()

        def dst_issue(c, b):
            pltpu.async_copy(dst_hbm.at[pl.ds(coff(c), chunk)], dsts[b], sds[b])

        def dst_wait(c, b):
            pltpu.make_async_copy(
                dst_hbm.at[pl.ds(coff(c), chunk)], dsts[b], sds[b]).wait()

        def gather_issue(b):
            pltpu.async_copy(hw_hbm.at[srcs[b]], gats[b], sgs[b])

        def gather_wait(b):
            pltpu.make_async_copy(hw_hbm.at[srcs[b]], gats[b], sgs[b]).wait()

        def scatter_issue(b):
            pltpu.async_copy(gats[b], acc_sh.at[dsts[b]], sss[b], add=True)

        def scatter_wait(b):
            pltpu.make_async_copy(gats[b], acc_sh.at[dsts[b]], sss[b]).wait()

        def compute(b):
            g_ref, e_ref = gats[b], ews[b]

            himask = jnp.int32(-65536)

            def edge(i, _):
                for g in range(hid // 32):
                    we_ = e_ref[i, pl.ds(g * 16, 16)]
                    ae = lax.bitcast_convert_type(
                        jnp.left_shift(we_, 16), jnp.float32)
                    be = lax.bitcast_convert_type(we_ & himask, jnp.float32)
                    slo = pl.ds(g * 32, 16)
                    shi = pl.ds(g * 32 + 16, 16)
                    g_ref[i, slo] = jnp.maximum(g_ref[i, slo] + ae, 0.0)
                    g_ref[i, shi] = jnp.maximum(g_ref[i, shi] + be, 0.0)
                return 0
            lax.fori_loop(0, chunk, edge, 0)

        for b in range(nbuf):
            pre_issue(b, b)
        dst_issue(0, 0)
        dst_issue(1, 1)

        z0 = gats[0]

        def zrow(i, _):
            for j in range(nvec):
                z0[i, pl.ds(j * 16, 16)] = jnp.zeros((16,), jnp.float32)
            return 0
        lax.fori_loop(0, chunk, zrow, 0)
        base_r = sid * rows_main

        def span_copy(nrows, fn):
            full, rem = divmod(nrows, chunk)
            for k in range(full):
                fn(k * chunk, chunk)
            if rem:
                fn(full * chunk, rem)

        def zero_fn(r0, cnt):
            pltpu.sync_copy(z0.at[pl.ds(0, cnt)],
                            acc_sh.at[pl.ds(base_r + r0, cnt)])

        @pl.when(sid < _NS - 1)
        def _():
            span_copy(rows_main, zero_fn)

        @pl.when(sid == _NS - 1)
        def _():
            span_copy(rows_last, zero_fn)

        plsc.subcore_barrier()

        pre_wait(0, 0)
        pre_wait(1, 1)
        pre_wait(2, 2)
        gather_issue(0)
        gather_issue(1)

        def quad(i, _):
            cb = nbuf * i
            for k in range(nbuf):
                c = cb + k
                b = k
                b2 = (k + 2) % nbuf
                b3 = (k + 3) % nbuf
                gather_wait(b)
                compute(b)
                dst_wait(c, b)
                scatter_issue(b)
                pre_issue(c + 4, b)
                pre_wait(c + 3, b3)
                if k < 2:
                    @pl.when(i > 0)
                    def _():
                        scatter_wait(b2)
                else:
                    scatter_wait(b2)
                dst_issue(c + 2, b2)
                gather_issue(b2)
            return 0

        lax.fori_loop(0, nquad, quad, 0)

        gather_wait(0)
        compute(0)
        dst_wait(niter - 2, 0)
        scatter_issue(0)
        scatter_wait(2)

        gather_wait(1)
        compute(1)
        dst_wait(niter - 1, 1)
        scatter_issue(1)
        scatter_wait(3)

        pre_wait(niter + 1, 3)
        scatter_wait(0)
        scatter_wait(1)
        plsc.subcore_barrier()

        def out_fn(r0, cnt):
            pltpu.sync_copy(acc_sh.at[pl.ds(base_r + r0, cnt)],
                            out_hbm.at[cid, pl.ds(base_r + r0, cnt)])

        @pl.when(sid < _NS - 1)
        def _():
            span_copy(rows_main, out_fn)

        @pl.when(sid == _NS - 1)
        def _():
            span_copy(rows_last, out_fn)

    return agg_kernel


def kernel(x, edge_index, edge_attr, batch, t, params):
    n, atom = x.shape
    e_cnt, bond = edge_attr.shape
    hid = params["W_node"].shape[1]
    g_cnt = t.shape[0]
    tdim = params["W_t1"].shape[0]
    nlayers = params["W_msg"].shape[0]

    blk_e = 2000
    blk_n = 2000

    f32 = jnp.float32
    b_node = params["b_node"].reshape(1, hid)
    b_edge = params["b_edge"].reshape(1, hid)
    blocks = np.arange(hid).reshape(hid // 32, 2, 16)
    perm = np.concatenate([blocks[:, 0, :].ravel(), blocks[:, 1, :].ravel()])
    w_msg_p = params["W_msg"][:, :, perm]
    b_msg_p = params["b_msg"][:, perm]
    b_upd = params["b_upd"]
    src = edge_index[0]
    dst = edge_index[1]
    batch2 = batch.reshape(n, 1)
    t2 = t.reshape(g_cnt, 1)

    edge_grid = e_cnt // blk_e
    ews = pl.pallas_call(
        _edge_body,
        grid=(edge_grid,),
        in_specs=[
            pl.BlockSpec((blk_e, bond), lambda i: (i, 0)),
            pl.BlockSpec((bond, hid), lambda i: (0, 0)),
            pl.BlockSpec((1, hid), lambda i: (0, 0)),
            pl.BlockSpec((nlayers, hid, hid), lambda i: (0, 0, 0)),
            pl.BlockSpec((nlayers, hid), lambda i: (0, 0)),
        ],
        out_specs=[pl.BlockSpec((blk_e, hid // 2), lambda i: (i, 0))] * 3,
        out_shape=[jax.ShapeDtypeStruct((e_cnt, hid // 2), jnp.int32)] * 3,
    )(edge_attr, params["W_edge"], b_edge, w_msg_p, b_msg_p)

    node_grid = n // blk_n
    h, hw = pl.pallas_call(
        _node_body,
        grid=(node_grid,),
        in_specs=[
            pl.BlockSpec((blk_n, atom), lambda i: (i, 0)),
            pl.BlockSpec((atom, hid), lambda i: (0, 0)),
            pl.BlockSpec((1, hid), lambda i: (0, 0)),
            pl.BlockSpec((hid, hid), lambda i: (0, 0)),
        ],
        out_specs=[pl.BlockSpec((blk_n, hid), lambda i: (i, 0))] * 2,
        out_shape=[jax.ShapeDtypeStruct((n, hid), f32)] * 2,
    )(x, params["W_node"], b_node, params["W_msg"][0])

    agg_kernel = _make_agg_kernel(n, e_cnt, hid)
    update_call = pl.pallas_call(
        _update_body,
        grid=(node_grid,),
        in_specs=[
            pl.BlockSpec((_NC, blk_n, hid), lambda i: (0, i, 0)),
            pl.BlockSpec((blk_n, hid), lambda i: (i, 0)),
            pl.BlockSpec((hid, hid), lambda i: (0, 0)),
            pl.BlockSpec((1, hid), lambda i: (0, 0)),
            pl.BlockSpec((hid, hid), lambda i: (0, 0)),
        ],
        out_specs=[pl.BlockSpec((blk_n, hid), lambda i: (i, 0))] * 2,
        out_shape=[jax.ShapeDtypeStruct((n, hid), f32)] * 2,
    )

    for l in range(nlayers - 1):
        agg2 = agg_kernel(hw, ews[l], src, dst)
        h, hw = update_call(
            agg2, h, params["W_upd"][l], b_upd[l].reshape(1, hid),
            params["W_msg"][l + 1],
        )

    agg2 = agg_kernel(hw, ews[nlayers - 1], src, dst)
    h, sums, cnts = pl.pallas_call(
        functools.partial(_update_pool_body, num_graphs=g_cnt),
        grid=(node_grid,),
        in_specs=[
            pl.BlockSpec((_NC, blk_n, hid), lambda i: (0, i, 0)),
            pl.BlockSpec((blk_n, hid), lambda i: (i, 0)),
            pl.BlockSpec((hid, hid), lambda i: (0, 0)),
            pl.BlockSpec((1, hid), lambda i: (0, 0)),
            pl.BlockSpec((blk_n, 1), lambda i: (i, 0)),
        ],
        out_specs=[
            pl.BlockSpec((blk_n, hid), lambda i: (i, 0)),
            pl.BlockSpec((g_cnt, hid), lambda i: (0, 0)),
            pl.BlockSpec((g_cnt, hid), lambda i: (0, 0)),
        ],
        out_shape=[
            jax.ShapeDtypeStruct((n, hid), f32),
            jax.ShapeDtypeStruct((g_cnt, hid), f32),
            jax.ShapeDtypeStruct((g_cnt, hid), f32),
        ],
        compiler_params=pltpu.CompilerParams(
            dimension_semantics=("arbitrary",)
        ),
    )(agg2, h, params["W_upd"][nlayers - 1],
      b_upd[nlayers - 1].reshape(1, hid), batch2)

    half = tdim // 2
    tg = pl.pallas_call(
        functools.partial(_graph_body, tdim=tdim),
        in_specs=[pl.BlockSpec(a.shape, lambda: tuple([0] * a.ndim)) for a in (
            t2, sums, cnts,
            params["W_t1"][:half], params["W_t1"][half:],
            params["b_t1"].reshape(1, hid), params["W_t2"],
            params["b_t2"].reshape(1, hid),
            params["W_g1"], params["b_g1"].reshape(1, hid),
            params["W_g2"], params["b_g2"].reshape(1, hid),
        )],
        out_specs=pl.BlockSpec((g_cnt, hid), lambda: (0, 0)),
        out_shape=jax.ShapeDtypeStruct((g_cnt, hid), f32),
    )(
        t2, sums, cnts,
        params["W_t1"][:half], params["W_t1"][half:],
        params["b_t1"].reshape(1, hid), params["W_t2"],
        params["b_t2"].reshape(1, hid),
        params["W_g1"], params["b_g1"].reshape(1, hid),
        params["W_g2"], params["b_g2"].reshape(1, hid),
    )

    out = pl.pallas_call(
        functools.partial(_final_body, num_graphs=g_cnt),
        grid=(node_grid,),
        in_specs=[
            pl.BlockSpec((blk_n, hid), lambda i: (i, 0)),
            pl.BlockSpec((blk_n, 1), lambda i: (i, 0)),
            pl.BlockSpec((g_cnt, hid), lambda i: (0, 0)),
            pl.BlockSpec((hid, hid), lambda i: (0, 0)),
            pl.BlockSpec((hid, hid), lambda i: (0, 0)),
            pl.BlockSpec((1, hid), lambda i: (0, 0)),
            pl.BlockSpec((hid, hid), lambda i: (0, 0)),
            pl.BlockSpec((1, hid), lambda i: (0, 0)),
            pl.BlockSpec((hid, atom), lambda i: (0, 0)),
            pl.BlockSpec((1, atom), lambda i: (0, 0)),
        ],
        out_specs=pl.BlockSpec((blk_n, atom), lambda i: (i, 0)),
        out_shape=jax.ShapeDtypeStruct((n, atom), f32),
    )(
        h, batch2, tg,
        params["W_n1"][:hid], params["W_n1"][hid:],
        params["b_n1"].reshape(1, hid),
        params["W_n2"], params["b_n2"].reshape(1, hid),
        params["W_n3"], params["b_n3"].reshape(1, atom),
    )
    return out
